# baseline (speedup 1.0000x reference)
import math

import jax
import jax.numpy as jnp
from jax.experimental import pallas as pl
from jax.experimental.pallas import tpu as pltpu

L = 4
D = 128
NH = 4
DH = D // NH
FF = 512
E = 8
NSH = 2
NC = 2
FLAT = 8 * 16 * 16
TP = 104
BB = 2
GRID = 16


def _mm(a, b):
    return jax.lax.dot_general(a, b, (((1,), (1,)), ((), ())),
                               preferred_element_type=jnp.float32)


def _mmn(a, b):
    return jax.lax.dot_general(a, b, (((1,), (0,)), ((), ())),
                               preferred_element_type=jnp.float32)


def _ln(x, g, b):
    m = jnp.mean(x, axis=-1, keepdims=True)
    v = jnp.mean((x - m) ** 2, axis=-1, keepdims=True)
    return (x - m) / jnp.sqrt(v + 1e-5) * g + b


def _gelu(x):
    return 0.5 * x * (1.0 + jax.lax.erf(x * (1.0 / math.sqrt(2.0))))


def _main_kernel(x_ref, ew1_ref, eb1_ref, ew2_ref, eb2_ref, pos_ref,
                 win_ref, bin_ref, wout_ref, bout_ref, g1_ref, b1_ref,
                 router_ref, w1_ref, wb1_ref, w2_ref, wb2_ref,
                 shr_ref, sw1_ref, sb1_ref, sw2_ref, sb2_ref,
                 g2_ref, b2_ref, clsw_ref, clsb_ref,
                 p_ref, cls_ref):
    M = BB * TP
    scale = 1.0 / math.sqrt(DH)

    xb = x_ref[...]
    eh = jnp.maximum(_mm(xb, ew1_ref[...]) + eb1_ref[...], 0.0)
    ez = jnp.maximum(_mm(eh, ew2_ref[...]) + eb2_ref[...], 0.0)
    pos = pos_ref[0]
    T = pos.shape[0]
    pad = jnp.zeros((TP - T, D), jnp.float32)
    parts = []
    for b in range(BB):
        parts.append(ez[b * T:(b + 1) * T] + pos)
        parts.append(pad)
    z = jnp.concatenate(parts, axis=0)
    kcol = jax.lax.broadcasted_iota(jnp.int32, (1, TP), 1)
    kmask = jnp.where(kcol % TP < 100, 0.0, -1e30)

    for l in range(L):
        qkv = _mm(z, win_ref[l]) + bin_ref[l:l + 1, :]
        obs = []
        for b in range(BB):
            qkvb = qkv[b * TP:(b + 1) * TP]
            cols = []
            for h in range(NH):
                q = qkvb[:, h * DH:(h + 1) * DH]
                k = qkvb[:, D + h * DH:D + (h + 1) * DH]
                v = qkvb[:, 2 * D + h * DH:2 * D + (h + 1) * DH]
                s = _mm(q, k) * scale + kmask
                s = jax.nn.softmax(s, axis=-1)
                cols.append(_mmn(s, v))
            obs.append(jnp.concatenate(cols, axis=1))
        o = jnp.concatenate(obs, axis=0)
        o = _mm(o, wout_ref[l]) + bout_ref[l:l + 1, :]
        z = _ln(z + o, g1_ref[l:l + 1, :], b1_ref[l:l + 1, :])

        logits = _mm(z, router_ref[l])
        probs = jax.nn.softmax(logits, axis=-1)
        p_ref[l] = probs

        iota = jax.lax.broadcasted_iota(jnp.int32, probs.shape, 1)
        i1 = jnp.argmax(probs, axis=-1, keepdims=True)
        m1 = jnp.max(probs, axis=-1, keepdims=True)
        masked = jnp.where(iota == i1, -jnp.inf, probs)
        i2 = jnp.argmax(masked, axis=-1, keepdims=True)
        m2 = jnp.max(masked, axis=-1, keepdims=True)
        wsum = m1 + m2
        comb = (jnp.where(iota == i1, m1 / wsum, 0.0)
                + jnp.where(iota == i2, m2 / wsum, 0.0))

        acc = jnp.zeros((M, D), jnp.float32)
        for e in range(E):
            h = _gelu(_mm(z, w1_ref[l, e]) + wb1_ref[l, e:e + 1, :])
            eo = _mm(h, w2_ref[l, e]) + wb2_ref[l, e:e + 1, :]
            acc = acc + comb[:, e:e + 1] * eo
        sp = jax.nn.softmax(_mm(z, shr_ref[l]), axis=-1)
        for e in range(NSH):
            h = _gelu(_mm(z, sw1_ref[l, e]) + sb1_ref[l, e:e + 1, :])
            so = _mm(h, sw2_ref[l, e]) + sb2_ref[l, e:e + 1, :]
            acc = acc + sp[:, e:e + 1] * so
        z = _ln(z + acc, g2_ref[l:l + 1, :], b2_ref[l:l + 1, :])

    r = jax.lax.broadcasted_iota(jnp.int32, (BB, M), 0)
    c = jax.lax.broadcasted_iota(jnp.int32, (BB, M), 1)
    P = jnp.where((c // TP == r) & (c % TP < 100), 0.01, 0.0)
    pooled = _mmn(P, z)
    cls_ref[0] = _mm(pooled, clsw_ref[...]) + clsb_ref[...]


def kernel(x, enc_w1, enc_b1, enc_w2, enc_b2, pos_embed, attn_in_w, attn_in_b,
           attn_out_w, attn_out_b, norm1_g, norm1_b, spec_router, spec_w1,
           spec_b1, spec_w2, spec_b2, sh_router, sh_w1, sh_b1, sh_w2, sh_b2,
           norm2_g, norm2_b, cls_w, cls_b):
    B, T = x.shape[0], x.shape[1]
    xf = x.reshape(B, T, FLAT)
    f32 = jnp.float32
    NP = B * TP

    full = lambda arr: pl.BlockSpec(arr.shape, lambda *_: (0,) * arr.ndim)

    enc_b1r = enc_b1.reshape(1, -1)
    enc_b2r = enc_b2.reshape(1, -1)
    xflat = xf.reshape(B * T, FLAT)
    MV = BB * T
    MB = BB * TP
    cls_br = cls_b.reshape(1, -1)

    probs_p, cls_o = pl.pallas_call(
        _main_kernel,
        grid=(GRID,),
        in_specs=[
            pl.BlockSpec((MV, FLAT), lambda i: (i, 0)),
            full(enc_w1), full(enc_b1r), full(enc_w2), full(enc_b2r),
            pl.BlockSpec((1, T, D), lambda i: (0, 0, 0)),
            full(attn_in_w), full(attn_in_b), full(attn_out_w),
            full(attn_out_b), full(norm1_g), full(norm1_b),
            full(spec_router), full(spec_w1), full(spec_b1),
            full(spec_w2), full(spec_b2),
            full(sh_router), full(sh_w1), full(sh_b1),
            full(sh_w2), full(sh_b2),
            full(norm2_g), full(norm2_b), full(cls_w), full(cls_br),
        ],
        out_specs=[
            pl.BlockSpec((L, MB, E), lambda i: (0, i, 0)),
            pl.BlockSpec((1, BB, NC), lambda i: (i, 0, 0)),
        ],
        out_shape=[
            jax.ShapeDtypeStruct((L, NP, E), f32),
            jax.ShapeDtypeStruct((GRID, BB, NC), f32),
        ],
        compiler_params=pltpu.CompilerParams(
            dimension_semantics=("parallel",)),
    )(xflat, enc_w1, enc_b1r, enc_w2, enc_b2r, pos_embed,
      attn_in_w, attn_in_b, attn_out_w, attn_out_b, norm1_g, norm1_b,
      spec_router, spec_w1, spec_b1, spec_w2, spec_b2,
      sh_router, sh_w1, sh_b1, sh_w2, sh_b2, norm2_g, norm2_b, cls_w, cls_br)

    cls = cls_o.reshape(B, NC)
    probs = probs_p.reshape(L, B, TP, E)[:, :, :T, :].reshape(L, B * T, E)
    return (cls,) + tuple(probs[l] for l in range(L))


# speedup vs baseline: 1.1800x; 1.1800x over previous
import math

import jax
import jax.numpy as jnp
from jax.experimental import pallas as pl
from jax.experimental.pallas import tpu as pltpu

L = 4
D = 128
NH = 4
DH = D // NH
FF = 512
E = 8
NSH = 2
NC = 2
FLAT = 8 * 16 * 16
TP = 104
BB = 2
GRID = 16


def _mm(a, b):
    return jax.lax.dot_general(a, b, (((1,), (1,)), ((), ())),
                               preferred_element_type=jnp.float32)


def _mmn(a, b):
    return jax.lax.dot_general(a, b, (((1,), (0,)), ((), ())),
                               preferred_element_type=jnp.float32)


def _ln(x, g, b):
    m = jnp.mean(x, axis=-1, keepdims=True)
    v = jnp.mean((x - m) ** 2, axis=-1, keepdims=True)
    return (x - m) / jnp.sqrt(v + 1e-5) * g + b


def _gelu(x):
    return 0.5 * x * (1.0 + jax.lax.erf(x * (1.0 / math.sqrt(2.0))))


def _enc_kernel(x_ref, w1_ref, b1_ref, w2_ref, b2_ref, pos_ref, o_ref):
    xb = x_ref[0]
    h = jnp.maximum(_mm(xb, w1_ref[...]) + b1_ref[...], 0.0)
    z = jnp.maximum(_mm(h, w2_ref[...]) + b2_ref[...], 0.0)
    z = z + pos_ref[0]
    o_ref[0] = jnp.concatenate(
        [z, jnp.zeros((TP - z.shape[0], D), jnp.float32)], axis=0)


def _main_kernel(z_ref, win_ref, bin_ref, wout_ref, bout_ref, g1_ref, b1_ref,
                 router_ref, w1_ref, wb1_ref, w2_ref, wb2_ref,
                 shr_ref, sw1_ref, sb1_ref, sw2_ref, sb2_ref,
                 g2_ref, b2_ref, clsw_ref, clsb_ref,
                 p_ref, cls_ref):
    z = z_ref[...]
    M = BB * TP
    scale = 1.0 / math.sqrt(DH)
    kcol4 = jax.lax.broadcasted_iota(jnp.int32, (1, NH * TP), 1)
    kmask4 = jnp.where(kcol4 % TP < 100, 0.0, -1e30)
    hrow = jax.lax.broadcasted_iota(jnp.int32, (NH * TP, NH), 0)
    hcolsel = jax.lax.broadcasted_iota(jnp.int32, (NH * TP, NH), 1)
    ones_bd = jnp.where(hrow // TP == hcolsel, 1.0, 0.0)

    for l in range(L):
        qkv = _mm(z, win_ref[l]) + bin_ref[l:l + 1, :]
        obs = []
        for b in range(BB):
            qkvb = qkv[b * TP:(b + 1) * TP]
            q = qkvb[:, 0:D]
            ks, vs = [], []
            for h in range(NH):
                kh = qkvb[:, D + h * DH:D + (h + 1) * DH]
                vh = qkvb[:, 2 * D + h * DH:2 * D + (h + 1) * DH]
                parts_k, parts_v = [], []
                if h > 0:
                    zl = jnp.zeros((TP, h * DH), jnp.float32)
                    parts_k.append(zl)
                    parts_v.append(zl)
                parts_k.append(kh)
                parts_v.append(vh)
                if h < NH - 1:
                    zr = jnp.zeros((TP, D - (h + 1) * DH), jnp.float32)
                    parts_k.append(zr)
                    parts_v.append(zr)
                ks.append(jnp.concatenate(parts_k, axis=1))
                vs.append(jnp.concatenate(parts_v, axis=1))
            kstack = jnp.concatenate(ks, axis=0)
            vstack = jnp.concatenate(vs, axis=0)
            s = _mm(q, kstack) * scale + kmask4
            s = jnp.exp(s - jnp.max(s, axis=-1, keepdims=True))
            den = _mmn(s, ones_bd)
            dinv = 1.0 / den
            dx = jnp.concatenate(
                [jnp.broadcast_to(dinv[:, h:h + 1], (TP, TP))
                 for h in range(NH)], axis=1)
            obs.append(_mmn(s * dx, vstack))
        o = jnp.concatenate(obs, axis=0)
        o = _mm(o, wout_ref[l]) + bout_ref[l:l + 1, :]
        z = _ln(z + o, g1_ref[l:l + 1, :], b1_ref[l:l + 1, :])

        logits = _mm(z, router_ref[l])
        probs = jax.nn.softmax(logits, axis=-1)
        p_ref[l] = probs

        iota = jax.lax.broadcasted_iota(jnp.int32, probs.shape, 1)
        i1 = jnp.argmax(probs, axis=-1, keepdims=True)
        m1 = jnp.max(probs, axis=-1, keepdims=True)
        masked = jnp.where(iota == i1, -jnp.inf, probs)
        i2 = jnp.argmax(masked, axis=-1, keepdims=True)
        m2 = jnp.max(masked, axis=-1, keepdims=True)
        wsum = m1 + m2
        comb = (jnp.where(iota == i1, m1 / wsum, 0.0)
                + jnp.where(iota == i2, m2 / wsum, 0.0))

        acc = jnp.zeros((M, D), jnp.float32)
        for e in range(E):
            h = _gelu(_mm(z, w1_ref[l, e]) + wb1_ref[l, e:e + 1, :])
            eo = _mm(h, w2_ref[l, e]) + wb2_ref[l, e:e + 1, :]
            acc = acc + comb[:, e:e + 1] * eo
        sp = jax.nn.softmax(_mm(z, shr_ref[l]), axis=-1)
        for e in range(NSH):
            h = _gelu(_mm(z, sw1_ref[l, e]) + sb1_ref[l, e:e + 1, :])
            so = _mm(h, sw2_ref[l, e]) + sb2_ref[l, e:e + 1, :]
            acc = acc + sp[:, e:e + 1] * so
        z = _ln(z + acc, g2_ref[l:l + 1, :], b2_ref[l:l + 1, :])

    r = jax.lax.broadcasted_iota(jnp.int32, (BB, M), 0)
    c = jax.lax.broadcasted_iota(jnp.int32, (BB, M), 1)
    P = jnp.where((c // TP == r) & (c % TP < 100), 0.01, 0.0)
    pooled = _mmn(P, z)
    cls_ref[0] = _mm(pooled, clsw_ref[...]) + clsb_ref[...]


def kernel(x, enc_w1, enc_b1, enc_w2, enc_b2, pos_embed, attn_in_w, attn_in_b,
           attn_out_w, attn_out_b, norm1_g, norm1_b, spec_router, spec_w1,
           spec_b1, spec_w2, spec_b2, sh_router, sh_w1, sh_b1, sh_w2, sh_b2,
           norm2_g, norm2_b, cls_w, cls_b):
    B, T = x.shape[0], x.shape[1]
    xf = x.reshape(B, T, FLAT)
    f32 = jnp.float32
    NP = B * TP

    full = lambda arr: pl.BlockSpec(arr.shape, lambda *_: (0,) * arr.ndim)

    enc_b1r = enc_b1.reshape(1, -1)
    enc_b2r = enc_b2.reshape(1, -1)
    zp = pl.pallas_call(
        _enc_kernel,
        grid=(B,),
        in_specs=[
            pl.BlockSpec((1, T, FLAT), lambda i: (i, 0, 0)),
            full(enc_w1), full(enc_b1r), full(enc_w2), full(enc_b2r),
            pl.BlockSpec((1, T, D), lambda i: (0, 0, 0)),
        ],
        out_specs=pl.BlockSpec((1, TP, D), lambda i: (i, 0, 0)),
        out_shape=jax.ShapeDtypeStruct((B, TP, D), f32),
        compiler_params=pltpu.CompilerParams(
            dimension_semantics=("parallel",)),
    )(xf, enc_w1, enc_b1r, enc_w2, enc_b2r, pos_embed)

    zf = zp.reshape(NP, D)
    MB = BB * TP
    cls_br = cls_b.reshape(1, -1)

    probs_p, cls_o = pl.pallas_call(
        _main_kernel,
        grid=(GRID,),
        in_specs=[
            pl.BlockSpec((MB, D), lambda i: (i, 0)),
            full(attn_in_w), full(attn_in_b), full(attn_out_w),
            full(attn_out_b), full(norm1_g), full(norm1_b),
            full(spec_router), full(spec_w1), full(spec_b1),
            full(spec_w2), full(spec_b2),
            full(sh_router), full(sh_w1), full(sh_b1),
            full(sh_w2), full(sh_b2),
            full(norm2_g), full(norm2_b), full(cls_w), full(cls_br),
        ],
        out_specs=[
            pl.BlockSpec((L, MB, E), lambda i: (0, i, 0)),
            pl.BlockSpec((1, BB, NC), lambda i: (i, 0, 0)),
        ],
        out_shape=[
            jax.ShapeDtypeStruct((L, NP, E), f32),
            jax.ShapeDtypeStruct((GRID, BB, NC), f32),
        ],
        compiler_params=pltpu.CompilerParams(
            dimension_semantics=("parallel",)),
    )(zf, attn_in_w, attn_in_b, attn_out_w, attn_out_b, norm1_g, norm1_b,
      spec_router, spec_w1, spec_b1, spec_w2, spec_b2,
      sh_router, sh_w1, sh_b1, sh_w2, sh_b2, norm2_g, norm2_b, cls_w, cls_br)

    cls = cls_o.reshape(B, NC)
    probs = probs_p.reshape(L, B, TP, E)[:, :, :T, :].reshape(L, B * T, E)
    return (cls,) + tuple(probs[l] for l in range(L))


# speedup vs baseline: 1.2380x; 1.0491x over previous
import math

import jax
import jax.numpy as jnp
from jax.experimental import pallas as pl
from jax.experimental.pallas import tpu as pltpu

L = 4
D = 128
NH = 4
DH = D // NH
FF = 512
E = 8
NSH = 2
NC = 2
FLAT = 8 * 16 * 16
TP = 104
BB = 2
GRID = 16


def _mm(a, b):
    return jax.lax.dot_general(a, b, (((1,), (1,)), ((), ())),
                               preferred_element_type=jnp.float32)


def _mmn(a, b):
    return jax.lax.dot_general(a, b, (((1,), (0,)), ((), ())),
                               preferred_element_type=jnp.float32)


def _ln(x, g, b):
    m = jnp.mean(x, axis=-1, keepdims=True)
    v = jnp.mean((x - m) ** 2, axis=-1, keepdims=True)
    return (x - m) / jnp.sqrt(v + 1e-5) * g + b


def _gelu(x):
    return 0.5 * x * (1.0 + jax.lax.erf(x * (1.0 / math.sqrt(2.0))))


def _enc_kernel(x_ref, w1_ref, b1_ref, w2_ref, b2_ref, pos_ref, o_ref):
    xb = x_ref[0]
    h = jnp.maximum(_mm(xb, w1_ref[...]) + b1_ref[...], 0.0)
    z = jnp.maximum(_mm(h, w2_ref[...]) + b2_ref[...], 0.0)
    z = z + pos_ref[0]
    o_ref[0] = jnp.concatenate(
        [z, jnp.zeros((TP - z.shape[0], D), jnp.float32)], axis=0)


def _main_kernel(z_ref, win_ref, bin_ref, wout_ref, bout_ref, g1_ref, b1_ref,
                 router_ref, w1_ref, wb1_ref, w2_ref, wb2_ref,
                 shr_ref, sw1_ref, sb1_ref, sw2_ref, sb2_ref,
                 g2_ref, b2_ref, clsw_ref, clsb_ref,
                 p_ref, cls_ref):
    z = z_ref[...]
    M = BB * TP
    scale = 1.0 / math.sqrt(DH)
    kcol = jax.lax.broadcasted_iota(jnp.int32, (1, TP), 1)
    kmask = jnp.where(kcol % TP < 100, 0.0, -1e30)

    for l in range(L):
        qkv = _mm(z, win_ref[l]) + bin_ref[l:l + 1, :]
        obs = []
        for b in range(BB):
            qkvb = qkv[b * TP:(b + 1) * TP]
            cols = []
            for h in range(NH):
                q = qkvb[:, h * DH:(h + 1) * DH]
                k = qkvb[:, D + h * DH:D + (h + 1) * DH]
                v = qkvb[:, 2 * D + h * DH:2 * D + (h + 1) * DH]
                s = _mm(q, k) * scale + kmask
                s = jax.nn.softmax(s, axis=-1)
                cols.append(_mmn(s, v))
            obs.append(jnp.concatenate(cols, axis=1))
        o = jnp.concatenate(obs, axis=0)
        o = _mm(o, wout_ref[l]) + bout_ref[l:l + 1, :]
        z = _ln(z + o, g1_ref[l:l + 1, :], b1_ref[l:l + 1, :])

        logits = _mm(z, router_ref[l])
        probs = jax.nn.softmax(logits, axis=-1)
        p_ref[l] = probs

        iota = jax.lax.broadcasted_iota(jnp.int32, probs.shape, 1)
        i1 = jnp.argmax(probs, axis=-1, keepdims=True)
        m1 = jnp.max(probs, axis=-1, keepdims=True)
        masked = jnp.where(iota == i1, -jnp.inf, probs)
        i2 = jnp.argmax(masked, axis=-1, keepdims=True)
        m2 = jnp.max(masked, axis=-1, keepdims=True)
        wsum = m1 + m2
        comb = (jnp.where(iota == i1, m1 / wsum, 0.0)
                + jnp.where(iota == i2, m2 / wsum, 0.0))

        acc = jnp.zeros((M, D), jnp.float32)
        for e in range(E):
            h = _gelu(_mm(z, w1_ref[l, e]) + wb1_ref[l, e:e + 1, :])
            eo = _mm(h, w2_ref[l, e]) + wb2_ref[l, e:e + 1, :]
            acc = acc + comb[:, e:e + 1] * eo
        sp = jax.nn.softmax(_mm(z, shr_ref[l]), axis=-1)
        for e in range(NSH):
            h = _gelu(_mm(z, sw1_ref[l, e]) + sb1_ref[l, e:e + 1, :])
            so = _mm(h, sw2_ref[l, e]) + sb2_ref[l, e:e + 1, :]
            acc = acc + sp[:, e:e + 1] * so
        z = _ln(z + acc, g2_ref[l:l + 1, :], b2_ref[l:l + 1, :])

    r = jax.lax.broadcasted_iota(jnp.int32, (BB, M), 0)
    c = jax.lax.broadcasted_iota(jnp.int32, (BB, M), 1)
    P = jnp.where((c // TP == r) & (c % TP < 100), 0.01, 0.0)
    pooled = _mmn(P, z)
    cls_ref[0] = _mm(pooled, clsw_ref[...]) + clsb_ref[...]


def kernel(x, enc_w1, enc_b1, enc_w2, enc_b2, pos_embed, attn_in_w, attn_in_b,
           attn_out_w, attn_out_b, norm1_g, norm1_b, spec_router, spec_w1,
           spec_b1, spec_w2, spec_b2, sh_router, sh_w1, sh_b1, sh_w2, sh_b2,
           norm2_g, norm2_b, cls_w, cls_b):
    B, T = x.shape[0], x.shape[1]
    xf = x.reshape(B, T, FLAT)
    f32 = jnp.float32
    NP = B * TP

    full = lambda arr: pl.BlockSpec(arr.shape, lambda *_: (0,) * arr.ndim)

    enc_b1r = enc_b1.reshape(1, -1)
    enc_b2r = enc_b2.reshape(1, -1)
    zp = pl.pallas_call(
        _enc_kernel,
        grid=(B,),
        in_specs=[
            pl.BlockSpec((1, T, FLAT), lambda i: (i, 0, 0)),
            full(enc_w1), full(enc_b1r), full(enc_w2), full(enc_b2r),
            pl.BlockSpec((1, T, D), lambda i: (0, 0, 0)),
        ],
        out_specs=pl.BlockSpec((1, TP, D), lambda i: (i, 0, 0)),
        out_shape=jax.ShapeDtypeStruct((B, TP, D), f32),
        compiler_params=pltpu.CompilerParams(
            dimension_semantics=("parallel",)),
    )(xf, enc_w1, enc_b1r, enc_w2, enc_b2r, pos_embed)

    zf = zp.reshape(NP, D)
    MB = BB * TP
    cls_br = cls_b.reshape(1, -1)

    probs_p, cls_o = pl.pallas_call(
        _main_kernel,
        grid=(GRID,),
        in_specs=[
            pl.BlockSpec((MB, D), lambda i: (i, 0)),
            full(attn_in_w), full(attn_in_b), full(attn_out_w),
            full(attn_out_b), full(norm1_g), full(norm1_b),
            full(spec_router), full(spec_w1), full(spec_b1),
            full(spec_w2), full(spec_b2),
            full(sh_router), full(sh_w1), full(sh_b1),
            full(sh_w2), full(sh_b2),
            full(norm2_g), full(norm2_b), full(cls_w), full(cls_br),
        ],
        out_specs=[
            pl.BlockSpec((L, MB, E), lambda i: (0, i, 0)),
            pl.BlockSpec((1, BB, NC), lambda i: (i, 0, 0)),
        ],
        out_shape=[
            jax.ShapeDtypeStruct((L, NP, E), f32),
            jax.ShapeDtypeStruct((GRID, BB, NC), f32),
        ],
        compiler_params=pltpu.CompilerParams(
            dimension_semantics=("parallel",)),
    )(zf, attn_in_w, attn_in_b, attn_out_w, attn_out_b, norm1_g, norm1_b,
      spec_router, spec_w1, spec_b1, spec_w2, spec_b2,
      sh_router, sh_w1, sh_b1, sh_w2, sh_b2, norm2_g, norm2_b, cls_w, cls_br)

    cls = cls_o.reshape(B, NC)
    probs = probs_p.reshape(L, B, TP, E)[:, :, :T, :].reshape(L, B * T, E)
    return (cls,) + tuple(probs[l] for l in range(L))
